# baseline (speedup 1.0000x reference)
import functools

import jax
import jax.numpy as jnp
from jax import lax
from jax.experimental import pallas as pl
from jax.experimental.pallas import tpu as pltpu
from jax.experimental.pallas import tpu_sc as plsc

_NC = 2
_NS = 16
_NW = _NC * _NS
_L = 16

_NUM_RELS = 10
_INV_DENOM = 1.0 / (_NUM_RELS - 1)


def _sigmoid_table(ability, w_relation, bias):
    n, d = ability.shape
    at = ability.T
    br = 8192

    def body(a_ref, w_ref, b_ref, o_ref):
        x = jnp.sum(a_ref[...] * w_ref[...], axis=0) + b_ref[0]
        o_ref[...] = jax.nn.sigmoid(x)

    return pl.pallas_call(
        body,
        grid=(-(-n // br),),
        in_specs=[
            pl.BlockSpec((d, br), lambda i: (0, i)),
            pl.BlockSpec((d, 1), lambda i: (0, 0)),
            pl.BlockSpec(memory_space=pltpu.SMEM),
        ],
        out_specs=pl.BlockSpec((br,), lambda i: (i,)),
        out_shape=jax.ShapeDtypeStruct((n,), jnp.float32),
    )(at, w_relation, bias)


def _blend(p16, t16):
    q = (1.0 - p16) * _INV_DENOM
    return p16 * t16 + q * (1.0 - t16)


@functools.lru_cache(maxsize=None)
def _edge_kernel(e, n_wkr, n_rel, n_tsk, chunk, n_per_tile, tail):
    n_lab = n_rel * n_tsk
    mesh = plsc.VectorSubcoreMesh(core_axis_name="c", subcore_axis_name="s")
    n_vec = chunk // _L
    unroll = next(u for u in (12, 8, 4, 2, 1) if n_vec % u == 0)
    n_tab = n_wkr + n_lab

    per_p = (n_wkr // _NS) & ~7
    per_row = (n_tsk // _NS) & ~7
    p_rem = n_wkr - _NS * per_p
    row_rem = n_tsk - _NS * per_row
    assert p_rem % 8 == 0 and row_rem % 8 == 0
    assert p_rem <= chunk and row_rem <= chunk

    vm_i = lambda: pltpu.VMEM((chunk,), jnp.int32)
    vm_f = lambda: pltpu.VMEM((chunk,), jnp.float32)
    scratch = [vm_i() for _ in range(6)] + [vm_f() for _ in range(4)] + \
              [pltpu.VMEM_SHARED((n_tab,), jnp.float32)] + \
              [pltpu.SemaphoreType.DMA for _ in range(8)]

    @functools.partial(
        pl.kernel,
        out_type=jax.ShapeDtypeStruct((1, e), jnp.float32),
        mesh=mesh,
        scratch_types=scratch,
        compiler_params=pltpu.CompilerParams(use_tc_tiling_on_sc=False),
    )
    def body(p_hbm, lab_hbm, wkr_hbm, tsk_hbm, rel_hbm, out_hbm,
             wkr0, wkr1, tsk0, tsk1, fid0, fid1,
             p0, p1, t0, t1, tab,
             si0, si1, sp0, sp1, st0, st1, so0, so1):
        wkr_b, tsk_b, fid_b = [wkr0, wkr1], [tsk0, tsk1], [fid0, fid1]
        p_b, t_b = [p0, p1], [t0, t1]
        sem_i, sem_p, sem_t, sem_o = [si0, si1], [sp0, sp1], [st0, st1], [so0, so1]

        sid = lax.axis_index("s")
        wid = sid * _NC + lax.axis_index("c")
        stage_sems = [sp0, sp1, st0, st1]

        bounce = [p0, p1, t0, t1]

        def seg_src(kind, so, w):
            if kind < 0:
                return p_hbm.at[pl.ds(so, w)]
            return lab_hbm.at[kind, pl.ds(so, w)]

        segs = []
        pos = 0
        while pos < per_p:
            w = min(chunk, per_p - pos)
            segs.append((-1, sid * per_p + pos, sid * per_p + pos, w))
            pos += w
        for r in range(n_rel):
            base = n_wkr + r * n_tsk
            pos = 0
            while pos < per_row:
                w = min(chunk, per_row - pos)
                segs.append((r, sid * per_row + pos,
                             base + sid * per_row + pos, w))
                pos += w
        n_main = len(segs)
        if p_rem:
            segs.append((-1, _NS * per_p, _NS * per_p, p_rem))
        if row_rem:
            for r in range(n_rel):
                segs.append((r, _NS * per_row,
                             n_wkr + r * n_tsk + _NS * per_row, row_rem))

        cps = {}
        for k, (kind, so, do, w) in enumerate(segs):
            rem = k >= n_main

            def issue(kind=kind, so=so, w=w, k=k):
                return pltpu.async_copy(
                    seg_src(kind, so, w), bounce[k % 4].at[pl.ds(0, w)],
                    stage_sems[k % 4])

            if rem:
                @pl.when(sid == 0)
                def _(issue=issue, k=k):
                    cps[k] = issue()
            else:
                cps[k] = issue()
            if k - 1 in cps:
                def hop2(k=k, prev=segs[k - 1]):
                    cp = cps.pop(k - 1)
                    cp.wait()
                    _, _, do1, w1 = prev
                    pltpu.sync_copy(bounce[(k - 1) % 4].at[pl.ds(0, w1)],
                                    tab.at[pl.ds(do1, w1)])
                if k - 1 >= n_main:
                    @pl.when(sid == 0)
                    def _(hop2=hop2):
                        hop2()
                else:
                    hop2()
        kl = len(segs) - 1

        def hop2_last():
            cp = cps.pop(kl)
            cp.wait()
            _, _, do1, w1 = segs[kl]
            pltpu.sync_copy(bounce[kl % 4].at[pl.ds(0, w1)],
                            tab.at[pl.ds(do1, w1)])

        if kl >= n_main:
            @pl.when(sid == 0)
            def _():
                hop2_last()
        else:
            hop2_last()

        def load_idx(g, b):
            off = (g * _NW) * chunk + wid * chunk
            return (pltpu.async_copy(wkr_hbm.at[pl.ds(off, chunk)], wkr_b[b], sem_i[b]),
                    pltpu.async_copy(tsk_hbm.at[pl.ds(off, chunk)], tsk_b[b], sem_i[b]),
                    pltpu.async_copy(rel_hbm.at[pl.ds(off, chunk)], fid_b[b], sem_i[b]))

        def fidx_loop(b):
            @plsc.parallel_loop(0, chunk, step=_L, unroll=unroll)
            def _(i):
                s = pl.ds(pl.multiple_of(i, _L), _L)
                fid_b[b][s] = fid_b[b][s] * (n_lab // _NUM_RELS) + tsk_b[b][s] + n_wkr

        def blend_loop(b):
            @plsc.parallel_loop(0, chunk, step=_L, unroll=unroll)
            def _(i):
                s = pl.ds(pl.multiple_of(i, _L), _L)
                p_b[b][s] = _blend(p_b[b][s], t_b[b][s])

        idx_cps = {0: load_idx(0, 0)}
        gat_cps = {}
        out_cps = {}
        for g in range(n_per_tile):
            b, nb = g % 2, (g + 1) % 2
            for cp in idx_cps.pop(g):
                cp.wait()
            fidx_loop(b)
            if g == 0:
                plsc.subcore_barrier()
            if g - 2 in out_cps:
                out_cps.pop(g - 2).wait()
            gat_cps[g] = (
                pltpu.async_copy(tab.at[wkr_b[b]], p_b[b], sem_p[b]),
                pltpu.async_copy(tab.at[fid_b[b]], t_b[b], sem_t[b]),
            )
            if g >= 1:
                for cp in gat_cps.pop(g - 1):
                    cp.wait()
            if g + 1 < n_per_tile:
                idx_cps[g + 1] = load_idx(g + 1, nb)
            if g >= 1:
                blend_loop(nb)
                off = ((g - 1) * _NW) * chunk + wid * chunk
                out_cps[g - 1] = pltpu.async_copy(
                    p_b[nb], out_hbm.at[0, pl.ds(off, chunk)], sem_o[nb])
        gl = n_per_tile - 1
        bl = gl % 2
        for cp in gat_cps.pop(gl):
            cp.wait()
        if gl - 1 in out_cps:
            out_cps.pop(gl - 1).wait()
        blend_loop(bl)
        off = (gl * _NW) * chunk + wid * chunk
        pltpu.sync_copy(p_b[bl], out_hbm.at[0, pl.ds(off, chunk)])

        if tail:
            t_off = n_per_tile * _NW * chunk
            t_per = -(-(tail // _L) // _NW) * _L
            n_full = tail // t_per
            t_rest = tail - n_full * t_per
            assert t_per % 8 == 0 and t_rest % _L == 0
            assert n_full <= _NW and (t_rest == 0 or n_full < _NW)

            def do_tail(my_off, width):
                sl = pl.ds(0, width)
                pltpu.sync_copy(wkr_hbm.at[pl.ds(my_off, width)], wkr0.at[sl])
                pltpu.sync_copy(tsk_hbm.at[pl.ds(my_off, width)], tsk0.at[sl])
                pltpu.sync_copy(rel_hbm.at[pl.ds(my_off, width)], fid0.at[sl])

                @plsc.parallel_loop(0, width, step=_L, unroll=1)
                def _(i):
                    s = pl.ds(pl.multiple_of(i, _L), _L)
                    fid0[s] = fid0[s] * (n_lab // _NUM_RELS) + tsk0[s] + n_wkr

                cp_p = pltpu.async_copy(tab.at[wkr0.at[sl]], p0.at[sl], sp0)
                cp_t = pltpu.async_copy(tab.at[fid0.at[sl]], t0.at[sl], st0)
                cp_p.wait()
                cp_t.wait()

                @plsc.parallel_loop(0, width, step=_L, unroll=1)
                def _(i):
                    s = pl.ds(pl.multiple_of(i, _L), _L)
                    p0[s] = _blend(p0[s], t0[s])
                pltpu.sync_copy(p0.at[sl], out_hbm.at[0, pl.ds(my_off, width)])

            @pl.when(wid < n_full)
            def _():
                do_tail(t_off + wid * t_per, t_per)
            if t_rest:
                @pl.when(wid == n_full)
                def _():
                    do_tail(t_off + n_full * t_per, t_rest)

    return body


def kernel(ability, labels, wkr_idx, rel_idx, tsk_idx, w_relation, bias):
    e = wkr_idx.shape[0]
    assert labels.shape[2] == _NUM_RELS

    n_tsk = labels.shape[0]
    n_wkr = ability.shape[0]
    p_tab = _sigmoid_table(ability, w_relation, bias)
    lab2 = labels.transpose(2, 1, 0).reshape(_NUM_RELS, n_tsk)

    chunk = 5184
    n_per_tile = e // (_NW * chunk)
    tail = e - _NW * chunk * n_per_tile
    assert n_per_tile >= 2 and tail < chunk and tail % _L == 0

    out = _edge_kernel(e, n_wkr, _NUM_RELS, n_tsk, chunk, n_per_tile, tail)(
        p_tab, lab2,
        wkr_idx.astype(jnp.int32), tsk_idx.astype(jnp.int32),
        rel_idx.astype(jnp.int32))
    return out.T


# speedup vs baseline: 1.0118x; 1.0118x over previous
import functools

import jax
import jax.numpy as jnp
from jax import lax
from jax.experimental import pallas as pl
from jax.experimental.pallas import tpu as pltpu
from jax.experimental.pallas import tpu_sc as plsc

_NC = 2
_NS = 16
_NW = _NC * _NS
_L = 16

_NUM_RELS = 10
_INV_DENOM = 1.0 / (_NUM_RELS - 1)


def _sigmoid_table(ability, w_relation, bias):
    n, d = ability.shape
    at = ability.T
    br = 8192

    def body(a_ref, w_ref, b_ref, o_ref):
        x = jnp.sum(a_ref[...] * w_ref[...], axis=0) + b_ref[0]
        o_ref[...] = jax.nn.sigmoid(x)

    return pl.pallas_call(
        body,
        grid=(-(-n // br),),
        in_specs=[
            pl.BlockSpec((d, br), lambda i: (0, i)),
            pl.BlockSpec((d, 1), lambda i: (0, 0)),
            pl.BlockSpec(memory_space=pltpu.SMEM),
        ],
        out_specs=pl.BlockSpec((br,), lambda i: (i,)),
        out_shape=jax.ShapeDtypeStruct((n,), jnp.float32),
    )(at, w_relation, bias)


def _blend(p16, t16):
    q = (1.0 - p16) * _INV_DENOM
    return p16 * t16 + q * (1.0 - t16)


@functools.lru_cache(maxsize=None)
def _edge_kernel(e, n_wkr, n_rel, n_tsk, chunk, n_per_tile, tail):
    n_lab = n_rel * n_tsk
    mesh = plsc.VectorSubcoreMesh(core_axis_name="c", subcore_axis_name="s")
    n_vec = chunk // _L
    unroll = next(u for u in (8, 4, 2, 1) if n_vec % u == 0)
    n_tab = n_wkr + n_lab

    per_p = (n_wkr // _NS) & ~7
    per_row = (n_tsk // _NS) & ~7
    p_rem = n_wkr - _NS * per_p
    row_rem = n_tsk - _NS * per_row
    assert p_rem % 8 == 0 and row_rem % 8 == 0
    assert p_rem <= chunk and row_rem <= chunk

    vm_i = lambda: pltpu.VMEM((chunk,), jnp.int32)
    vm_f = lambda: pltpu.VMEM((chunk,), jnp.float32)
    scratch = [vm_i() for _ in range(6)] + [vm_f() for _ in range(4)] + \
              [pltpu.VMEM_SHARED((n_tab,), jnp.float32)] + \
              [pltpu.SemaphoreType.DMA for _ in range(8)]

    @functools.partial(
        pl.kernel,
        out_type=jax.ShapeDtypeStruct((1, e), jnp.float32),
        mesh=mesh,
        scratch_types=scratch,
        compiler_params=pltpu.CompilerParams(use_tc_tiling_on_sc=False),
    )
    def body(p_hbm, lab_hbm, wkr_hbm, tsk_hbm, rel_hbm, out_hbm,
             wkr0, wkr1, tsk0, tsk1, fid0, fid1,
             p0, p1, t0, t1, tab,
             si0, si1, sp0, sp1, st0, st1, so0, so1):
        wkr_b, tsk_b, fid_b = [wkr0, wkr1], [tsk0, tsk1], [fid0, fid1]
        p_b, t_b = [p0, p1], [t0, t1]
        sem_i, sem_p, sem_t, sem_o = [si0, si1], [sp0, sp1], [st0, st1], [so0, so1]

        sid = lax.axis_index("s")
        wid = sid * _NC + lax.axis_index("c")
        stage_sems = [sp0, sp1, st0, st1]

        def load_idx(g, b):
            off = (g * _NW) * chunk + wid * chunk
            return (pltpu.async_copy(wkr_hbm.at[pl.ds(off, chunk)], wkr_b[b], sem_i[b]),
                    pltpu.async_copy(tsk_hbm.at[pl.ds(off, chunk)], tsk_b[b], sem_i[b]),
                    pltpu.async_copy(rel_hbm.at[pl.ds(off, chunk)], fid_b[b], sem_i[b]))

        idx_cps = {0: load_idx(0, 0)}
        if n_per_tile > 1:
            idx_cps[1] = load_idx(1, 1)

        bounce = [p0, p1, t0, t1]

        def seg_src(kind, so, w):
            if kind < 0:
                return p_hbm.at[pl.ds(so, w)]
            return lab_hbm.at[kind, pl.ds(so, w)]

        segs = []
        pos = 0
        while pos < per_p:
            w = min(chunk, per_p - pos)
            segs.append((-1, sid * per_p + pos, sid * per_p + pos, w))
            pos += w
        for r in range(n_rel):
            base = n_wkr + r * n_tsk
            pos = 0
            while pos < per_row:
                w = min(chunk, per_row - pos)
                segs.append((r, sid * per_row + pos,
                             base + sid * per_row + pos, w))
                pos += w
        n_main = len(segs)
        if p_rem:
            segs.append((-1, _NS * per_p, _NS * per_p, p_rem))
        if row_rem:
            for r in range(n_rel):
                segs.append((r, _NS * per_row,
                             n_wkr + r * n_tsk + _NS * per_row, row_rem))

        cps = {}
        for k, (kind, so, do, w) in enumerate(segs):
            rem = k >= n_main

            def issue(kind=kind, so=so, w=w, k=k):
                return pltpu.async_copy(
                    seg_src(kind, so, w), bounce[k % 4].at[pl.ds(0, w)],
                    stage_sems[k % 4])

            if rem:
                @pl.when(sid == 0)
                def _(issue=issue, k=k):
                    cps[k] = issue()
            else:
                cps[k] = issue()
            if k - 1 in cps:
                def hop2(k=k, prev=segs[k - 1]):
                    cp = cps.pop(k - 1)
                    cp.wait()
                    _, _, do1, w1 = prev
                    pltpu.sync_copy(bounce[(k - 1) % 4].at[pl.ds(0, w1)],
                                    tab.at[pl.ds(do1, w1)])
                if k - 1 >= n_main:
                    @pl.when(sid == 0)
                    def _(hop2=hop2):
                        hop2()
                else:
                    hop2()
        kl = len(segs) - 1

        def hop2_last():
            cp = cps.pop(kl)
            cp.wait()
            _, _, do1, w1 = segs[kl]
            pltpu.sync_copy(bounce[kl % 4].at[pl.ds(0, w1)],
                            tab.at[pl.ds(do1, w1)])

        if kl >= n_main:
            @pl.when(sid == 0)
            def _():
                hop2_last()
        else:
            hop2_last()

        def fidx_loop(b):
            @plsc.parallel_loop(0, chunk, step=_L, unroll=unroll)
            def _(i):
                s = pl.ds(pl.multiple_of(i, _L), _L)
                fid_b[b][s] = fid_b[b][s] * (n_lab // _NUM_RELS) + tsk_b[b][s] + n_wkr

        def blend_loop(b):
            @plsc.parallel_loop(0, chunk, step=_L, unroll=unroll)
            def _(i):
                s = pl.ds(pl.multiple_of(i, _L), _L)
                p_b[b][s] = _blend(p_b[b][s], t_b[b][s])

        gat_cps = {}
        out_cps = {}
        for g in range(n_per_tile):
            b, nb = g % 2, (g + 1) % 2
            for cp in idx_cps.pop(g):
                cp.wait()
            fidx_loop(b)
            if g == 0:
                plsc.subcore_barrier()
            if g - 2 in out_cps:
                out_cps.pop(g - 2).wait()
            gat_cps[g] = (
                pltpu.async_copy(tab.at[wkr_b[b]], p_b[b], sem_p[b]),
                pltpu.async_copy(tab.at[fid_b[b]], t_b[b], sem_t[b]),
            )
            if g >= 1:
                for cp in gat_cps.pop(g - 1):
                    cp.wait()
            if g + 1 < n_per_tile and g + 1 not in idx_cps:
                idx_cps[g + 1] = load_idx(g + 1, nb)
            if g >= 1:
                blend_loop(nb)
                off = ((g - 1) * _NW) * chunk + wid * chunk
                out_cps[g - 1] = pltpu.async_copy(
                    p_b[nb], out_hbm.at[0, pl.ds(off, chunk)], sem_o[nb])
        gl = n_per_tile - 1
        bl = gl % 2
        for cp in gat_cps.pop(gl):
            cp.wait()
        if gl - 1 in out_cps:
            out_cps.pop(gl - 1).wait()
        blend_loop(bl)
        off = (gl * _NW) * chunk + wid * chunk
        pltpu.sync_copy(p_b[bl], out_hbm.at[0, pl.ds(off, chunk)])

        if tail:
            t_off = n_per_tile * _NW * chunk
            t_per = -(-(tail // _L) // _NW) * _L
            n_full = tail // t_per
            t_rest = tail - n_full * t_per
            assert t_per % 8 == 0 and t_rest % _L == 0
            assert n_full <= _NW and (t_rest == 0 or n_full < _NW)

            def do_tail(my_off, width):
                sl = pl.ds(0, width)
                pltpu.sync_copy(wkr_hbm.at[pl.ds(my_off, width)], wkr0.at[sl])
                pltpu.sync_copy(tsk_hbm.at[pl.ds(my_off, width)], tsk0.at[sl])
                pltpu.sync_copy(rel_hbm.at[pl.ds(my_off, width)], fid0.at[sl])

                @plsc.parallel_loop(0, width, step=_L, unroll=1)
                def _(i):
                    s = pl.ds(pl.multiple_of(i, _L), _L)
                    fid0[s] = fid0[s] * (n_lab // _NUM_RELS) + tsk0[s] + n_wkr

                cp_p = pltpu.async_copy(tab.at[wkr0.at[sl]], p0.at[sl], sp0)
                cp_t = pltpu.async_copy(tab.at[fid0.at[sl]], t0.at[sl], st0)
                cp_p.wait()
                cp_t.wait()

                @plsc.parallel_loop(0, width, step=_L, unroll=1)
                def _(i):
                    s = pl.ds(pl.multiple_of(i, _L), _L)
                    p0[s] = _blend(p0[s], t0[s])
                pltpu.sync_copy(p0.at[sl], out_hbm.at[0, pl.ds(my_off, width)])

            @pl.when(wid < n_full)
            def _():
                do_tail(t_off + wid * t_per, t_per)
            if t_rest:
                @pl.when(wid == n_full)
                def _():
                    do_tail(t_off + n_full * t_per, t_rest)

    return body


def kernel(ability, labels, wkr_idx, rel_idx, tsk_idx, w_relation, bias):
    e = wkr_idx.shape[0]
    assert labels.shape[2] == _NUM_RELS

    n_tsk = labels.shape[0]
    n_wkr = ability.shape[0]
    p_tab = _sigmoid_table(ability, w_relation, bias)
    lab2 = labels.transpose(2, 1, 0).reshape(_NUM_RELS, n_tsk)

    chunk = 5184
    n_per_tile = e // (_NW * chunk)
    tail = e - _NW * chunk * n_per_tile
    assert n_per_tile >= 2 and tail < chunk and tail % _L == 0

    out = _edge_kernel(e, n_wkr, _NUM_RELS, n_tsk, chunk, n_per_tile, tail)(
        p_tab, lab2,
        wkr_idx.astype(jnp.int32), tsk_idx.astype(jnp.int32),
        rel_idx.astype(jnp.int32))
    return out.T


# speedup vs baseline: 1.0437x; 1.0315x over previous
import functools

import jax
import jax.numpy as jnp
from jax import lax
from jax.experimental import pallas as pl
from jax.experimental.pallas import tpu as pltpu
from jax.experimental.pallas import tpu_sc as plsc

_NC = 2
_NS = 16
_NW = _NC * _NS
_L = 16

_NUM_RELS = 10
_INV_DENOM = 1.0 / (_NUM_RELS - 1)
_SC_PARAMS = pltpu.CompilerParams(use_tc_tiling_on_sc=False)


def _sigmoid_table(ability, w_relation, bias):
    n, d = ability.shape
    at = ability.T
    br = 8192

    def body(a_ref, w_ref, b_ref, o_ref):
        x = jnp.sum(a_ref[...] * w_ref[...], axis=0) + b_ref[0]
        o_ref[...] = jax.nn.sigmoid(x)

    return pl.pallas_call(
        body,
        grid=(-(-n // br),),
        in_specs=[
            pl.BlockSpec((d, br), lambda i: (0, i)),
            pl.BlockSpec((d, 1), lambda i: (0, 0)),
            pl.BlockSpec(memory_space=pltpu.SMEM),
        ],
        out_specs=pl.BlockSpec((br,), lambda i: (i,)),
        out_shape=jax.ShapeDtypeStruct((n,), jnp.float32),
    )(at, w_relation, bias)


def _blend(p16, t16):
    q = (1.0 - p16) * _INV_DENOM
    return p16 * t16 + q * (1.0 - t16)


def _tail_split(tail):
    t_per = -(-(tail // _L) // _NW) * _L
    n_full = tail // t_per
    t_rest = tail - n_full * t_per
    assert t_per % 8 == 0 and t_rest % _L == 0
    assert n_full <= _NW and (t_rest == 0 or n_full < _NW)
    return t_per, n_full, t_rest


@functools.lru_cache(maxsize=None)
def _p_gather_kernel(e, n_wkr, chunk, n_per_tile, tail):
    mesh = plsc.VectorSubcoreMesh(core_axis_name="c", subcore_axis_name="s")
    per_p = (n_wkr // _NS) & ~7
    p_rem = n_wkr - _NS * per_p
    assert per_p <= chunk and p_rem % 8 == 0 and p_rem <= chunk

    scratch = [pltpu.VMEM((chunk,), jnp.int32) for _ in range(2)] + \
              [pltpu.VMEM((chunk,), jnp.float32) for _ in range(2)] + \
              [pltpu.VMEM_SHARED((n_wkr,), jnp.float32)] + \
              [pltpu.SemaphoreType.DMA for _ in range(6)]

    @functools.partial(
        pl.kernel,
        out_type=jax.ShapeDtypeStruct((e,), jnp.float32),
        mesh=mesh,
        scratch_types=scratch,
        compiler_params=_SC_PARAMS,
    )
    def body(p_hbm, wkr_hbm, out_hbm, wkr0, wkr1, g0, g1, tab,
             si0, si1, sg0, sg1, so0, so1):
        wkr_b, g_b = [wkr0, wkr1], [g0, g1]
        sem_i, sem_g, sem_o = [si0, si1], [sg0, sg1], [so0, so1]
        sid = lax.axis_index("s")
        wid = sid * _NC + lax.axis_index("c")

        def load_idx(g, b):
            off = (g * _NW) * chunk + wid * chunk
            return pltpu.async_copy(wkr_hbm.at[pl.ds(off, chunk)],
                                    wkr_b[b], sem_i[b])

        idx_cps = {0: load_idx(0, 0)}
        if n_per_tile > 1:
            idx_cps[1] = load_idx(1, 1)

        cp = pltpu.async_copy(p_hbm.at[pl.ds(sid * per_p, per_p)],
                              g0.at[pl.ds(0, per_p)], sg0)
        cp.wait()
        pltpu.sync_copy(g0.at[pl.ds(0, per_p)],
                        tab.at[pl.ds(sid * per_p, per_p)])
        if p_rem:
            @pl.when(sid == 0)
            def _():
                cpr = pltpu.async_copy(p_hbm.at[pl.ds(_NS * per_p, p_rem)],
                                       g0.at[pl.ds(0, p_rem)], sg0)
                cpr.wait()
                pltpu.sync_copy(g0.at[pl.ds(0, p_rem)],
                                tab.at[pl.ds(_NS * per_p, p_rem)])

        gat_cps = {}
        out_cps = {}
        for g in range(n_per_tile):
            b, nb = g % 2, (g + 1) % 2
            idx_cps.pop(g).wait()
            if g == 0:
                plsc.subcore_barrier()
            if g - 2 in out_cps:
                out_cps.pop(g - 2).wait()
            gat_cps[g] = pltpu.async_copy(tab.at[wkr_b[b]], g_b[b], sem_g[b])
            if g >= 1:
                gat_cps.pop(g - 1).wait()
            if g + 1 < n_per_tile and g + 1 not in idx_cps:
                idx_cps[g + 1] = load_idx(g + 1, nb)
            if g >= 1:
                off = ((g - 1) * _NW) * chunk + wid * chunk
                out_cps[g - 1] = pltpu.async_copy(
                    g_b[nb], out_hbm.at[pl.ds(off, chunk)], sem_o[nb])
        gl = n_per_tile - 1
        bl = gl % 2
        gat_cps.pop(gl).wait()
        if gl - 1 in out_cps:
            out_cps.pop(gl - 1).wait()
        off = (gl * _NW) * chunk + wid * chunk
        pltpu.sync_copy(g_b[bl], out_hbm.at[pl.ds(off, chunk)])

        if tail:
            t_off = n_per_tile * _NW * chunk
            t_per, n_full, t_rest = _tail_split(tail)

            def do_tail(my_off, width):
                sl = pl.ds(0, width)
                pltpu.sync_copy(wkr_hbm.at[pl.ds(my_off, width)], wkr0.at[sl])
                cpt = pltpu.async_copy(tab.at[wkr0.at[sl]], g0.at[sl], sg0)
                cpt.wait()
                pltpu.sync_copy(g0.at[sl], out_hbm.at[pl.ds(my_off, width)])

            @pl.when(wid < n_full)
            def _():
                do_tail(t_off + wid * t_per, t_per)
            if t_rest:
                @pl.when(wid == n_full)
                def _():
                    do_tail(t_off + n_full * t_per, t_rest)

    return body


@functools.lru_cache(maxsize=None)
def _t_blend_kernel(e, n_rel, n_tsk, chunk, n_per_tile, tail):
    mesh = plsc.VectorSubcoreMesh(core_axis_name="c", subcore_axis_name="s")
    n_lab = n_rel * n_tsk
    n_vec = chunk // _L
    unroll = next(u for u in (8, 4, 2, 1) if n_vec % u == 0)
    per_row = (n_tsk // _NS) & ~7
    row_rem = n_tsk - _NS * per_row
    assert per_row <= chunk and row_rem % 8 == 0 and row_rem <= chunk

    scratch = [pltpu.VMEM((chunk,), jnp.int32) for _ in range(4)] + \
              [pltpu.VMEM((chunk,), jnp.float32) for _ in range(4)] + \
              [pltpu.VMEM_SHARED((n_lab,), jnp.float32)] + \
              [pltpu.SemaphoreType.DMA for _ in range(8)]

    @functools.partial(
        pl.kernel,
        out_type=jax.ShapeDtypeStruct((1, e), jnp.float32),
        mesh=mesh,
        scratch_types=scratch,
        compiler_params=_SC_PARAMS,
    )
    def body(lab_hbm, pa_hbm, tsk_hbm, rel_hbm, out_hbm,
             tsk0, tsk1, fid0, fid1, pa0, pa1, t0, t1, tab,
             si0, si1, sp0, sp1, st0, st1, so0, so1):
        tsk_b, fid_b = [tsk0, tsk1], [fid0, fid1]
        pa_b, t_b = [pa0, pa1], [t0, t1]
        sem_i, sem_t, sem_o = [si0, si1], [st0, st1], [so0, so1]
        stage_sems = [sp0, sp1]
        sid = lax.axis_index("s")
        wid = sid * _NC + lax.axis_index("c")

        def load_idx(g, b):
            off = (g * _NW) * chunk + wid * chunk
            return (pltpu.async_copy(tsk_hbm.at[pl.ds(off, chunk)], tsk_b[b], sem_i[b]),
                    pltpu.async_copy(rel_hbm.at[pl.ds(off, chunk)], fid_b[b], sem_i[b]),
                    pltpu.async_copy(pa_hbm.at[pl.ds(off, chunk)], pa_b[b], sem_i[b]))

        idx_cps = {0: load_idx(0, 0)}
        if n_per_tile > 1:
            idx_cps[1] = load_idx(1, 1)

        bounce = [t0, t1]
        cps = {}
        segs = [(r, sid * per_row, r * n_tsk + sid * per_row, per_row)
                for r in range(n_rel)]
        n_main = len(segs)
        if row_rem:
            for r in range(n_rel):
                segs.append((r, _NS * per_row, r * n_tsk + _NS * per_row,
                             row_rem))

        def hop2(kk):
            cps.pop(kk).wait()
            _, _, do1, w1 = segs[kk]
            pltpu.sync_copy(bounce[kk % 2].at[pl.ds(0, w1)],
                            tab.at[pl.ds(do1, w1)])

        for k, (r, so, do, w) in enumerate(segs):
            def issue(r=r, so=so, w=w, k=k):
                cps[k] = pltpu.async_copy(lab_hbm.at[r, pl.ds(so, w)],
                                          bounce[k % 2].at[pl.ds(0, w)],
                                          stage_sems[k % 2])

            if k >= n_main:
                pl.when(sid == 0)(issue)
            else:
                issue()
            if k - 1 in cps:
                if k - 1 >= n_main:
                    pl.when(sid == 0)(functools.partial(hop2, k - 1))
                else:
                    hop2(k - 1)
        kl = len(segs) - 1
        if kl in cps:
            if kl >= n_main:
                pl.when(sid == 0)(functools.partial(hop2, kl))
            else:
                hop2(kl)

        def fidx_loop(b):
            @plsc.parallel_loop(0, chunk, step=_L, unroll=unroll)
            def _(i):
                s = pl.ds(pl.multiple_of(i, _L), _L)
                fid_b[b][s] = fid_b[b][s] * n_tsk + tsk_b[b][s]

        def blend_loop(b):
            @plsc.parallel_loop(0, chunk, step=_L, unroll=unroll)
            def _(i):
                s = pl.ds(pl.multiple_of(i, _L), _L)
                pa_b[b][s] = _blend(pa_b[b][s], t_b[b][s])

        gat_cps = {}
        out_cps = {}
        for g in range(n_per_tile):
            b, nb = g % 2, (g + 1) % 2
            for cp in idx_cps.pop(g):
                cp.wait()
            fidx_loop(b)
            if g == 0:
                plsc.subcore_barrier()
            gat_cps[g] = pltpu.async_copy(tab.at[fid_b[b]], t_b[b], sem_t[b])
            if g >= 1:
                gat_cps.pop(g - 1).wait()
            if g + 1 < n_per_tile and g + 1 not in idx_cps:
                idx_cps[g + 1] = load_idx(g + 1, nb)
            if g >= 1:
                if g - 1 in out_cps:
                    out_cps.pop(g - 1).wait()
                blend_loop(nb)
                off = ((g - 1) * _NW) * chunk + wid * chunk
                out_cps[g - 1] = pltpu.async_copy(
                    pa_b[nb], out_hbm.at[0, pl.ds(off, chunk)], sem_o[nb])
        gl = n_per_tile - 1
        bl = gl % 2
        gat_cps.pop(gl).wait()
        if gl - 1 in out_cps:
            out_cps.pop(gl - 1).wait()
        blend_loop(bl)
        off = (gl * _NW) * chunk + wid * chunk
        pltpu.sync_copy(pa_b[bl], out_hbm.at[0, pl.ds(off, chunk)])

        if tail:
            t_off = n_per_tile * _NW * chunk
            t_per, n_full, t_rest = _tail_split(tail)

            def do_tail(my_off, width):
                sl = pl.ds(0, width)
                pltpu.sync_copy(tsk_hbm.at[pl.ds(my_off, width)], tsk0.at[sl])
                pltpu.sync_copy(rel_hbm.at[pl.ds(my_off, width)], fid0.at[sl])
                pltpu.sync_copy(pa_hbm.at[pl.ds(my_off, width)], pa0.at[sl])

                @plsc.parallel_loop(0, width, step=_L, unroll=1)
                def _(i):
                    s = pl.ds(pl.multiple_of(i, _L), _L)
                    fid0[s] = fid0[s] * n_tsk + tsk0[s]

                cpt = pltpu.async_copy(tab.at[fid0.at[sl]], t0.at[sl], st0)
                cpt.wait()

                @plsc.parallel_loop(0, width, step=_L, unroll=1)
                def _(i):
                    s = pl.ds(pl.multiple_of(i, _L), _L)
                    pa0[s] = _blend(pa0[s], t0[s])
                pltpu.sync_copy(pa0.at[sl], out_hbm.at[0, pl.ds(my_off, width)])

            @pl.when(wid < n_full)
            def _():
                do_tail(t_off + wid * t_per, t_per)
            if t_rest:
                @pl.when(wid == n_full)
                def _():
                    do_tail(t_off + n_full * t_per, t_rest)

    return body


def kernel(ability, labels, wkr_idx, rel_idx, tsk_idx, w_relation, bias):
    e = wkr_idx.shape[0]
    assert labels.shape[2] == _NUM_RELS
    n_tsk = labels.shape[0]
    n_wkr = ability.shape[0]

    p_tab = _sigmoid_table(ability, w_relation, bias)
    lab2 = labels.transpose(2, 1, 0).reshape(_NUM_RELS, n_tsk)

    c1 = 15616
    n1 = e // (_NW * c1)
    tail1 = e - _NW * c1 * n1
    assert n1 >= 2 and tail1 < c1 and tail1 % _L == 0
    p_all = _p_gather_kernel(e, n_wkr, c1, n1, tail1)(
        p_tab, wkr_idx.astype(jnp.int32))

    c2 = 7808
    n2 = e // (_NW * c2)
    tail2 = e - _NW * c2 * n2
    assert n2 >= 2 and tail2 < c2 and tail2 % _L == 0
    out = _t_blend_kernel(e, _NUM_RELS, n_tsk, c2, n2, tail2)(
        lab2, p_all, tsk_idx.astype(jnp.int32), rel_idx.astype(jnp.int32))
    return out.T
